# baseline (device time: 409336 ns/iter reference)
import jax
import jax.numpy as jnp
from jax import lax
from jax.experimental import pallas as pl
from jax.experimental.pallas import tpu as pltpu

STAGE_MIB = 8


def kernel(x):
    m, n = x.shape
    stage_rows = STAGE_MIB * 1024 * 1024 // (n * 4)
    assert m % stage_rows == 0
    n_stage = m // stage_rows

    def body(x_ref, out_ref, stage_buf, ld_sems, st_sems, send_sem, recv_sem):
        my_x = lax.axis_index("x")
        my_y = lax.axis_index("y")
        my_z = lax.axis_index("z")
        partner = (my_x, my_y, 1 - my_z)

        barrier_sem = pltpu.get_barrier_semaphore()
        pl.semaphore_signal(
            barrier_sem,
            inc=1,
            device_id=partner,
            device_id_type=pl.DeviceIdType.MESH,
        )
        pl.semaphore_wait(barrier_sem, 1)

        rdma = pltpu.make_async_remote_copy(
            src_ref=x_ref,
            dst_ref=out_ref.at[pl.ds(my_z * m, m), :],
            send_sem=send_sem,
            recv_sem=recv_sem,
            device_id=partner,
            device_id_type=pl.DeviceIdType.MESH,
        )
        rdma.start()

        lds = [
            pltpu.make_async_copy(
                x_ref.at[pl.ds(s * stage_rows, stage_rows), :],
                stage_buf.at[s % 2],
                ld_sems.at[s % 2],
            )
            for s in range(n_stage)
        ]
        sts = []
        lds[0].start()
        for s in range(n_stage):
            lds[s].wait()
            st = pltpu.make_async_copy(
                stage_buf.at[s % 2],
                out_ref.at[pl.ds(my_z * m + s * stage_rows, stage_rows), :],
                st_sems.at[s % 2],
            )
            st.start()
            sts.append(st)
            if s + 1 < n_stage:
                if s >= 1:
                    sts[s - 1].wait()
                lds[s + 1].start()
        for st in sts[max(0, n_stage - 2):]:
            st.wait()

        rdma.wait()

    return pl.pallas_call(
        body,
        out_shape=jax.ShapeDtypeStruct((2 * m, n), x.dtype),
        in_specs=[pl.BlockSpec(memory_space=pl.ANY)],
        out_specs=pl.BlockSpec(memory_space=pl.ANY),
        scratch_shapes=[
            pltpu.VMEM((2, stage_rows, n), x.dtype),
            pltpu.SemaphoreType.DMA((2,)),
            pltpu.SemaphoreType.DMA((2,)),
            pltpu.SemaphoreType.DMA,
            pltpu.SemaphoreType.DMA,
        ],
        compiler_params=pltpu.CompilerParams(collective_id=0),
    )(x)
